# baseline (speedup 1.0000x reference)
import functools

import jax
import jax.numpy as jnp
from jax import lax
from jax.experimental import pallas as pl
from jax.experimental.pallas import tpu as pltpu
from jax.experimental.pallas import tpu_sc as plsc

_EPS = 1e-12


def _tc_assign_body(z_ref, w_ref, cb_ref, zr_ref, idx_ref, fidx_ref, cn_ref):
    i = pl.program_id(0)
    M = 4
    K = 1024
    Ds = 64

    @pl.when(i == 0)
    def _():
        cb = cb_ref[...]
        cn_ref[...] = cb / jnp.maximum(
            jnp.sqrt(jnp.sum(cb * cb, axis=-1, keepdims=True)), _EPS)

    z = z_ref[...]
    w = w_ref[...]
    zr = jnp.dot(z, w, preferred_element_type=jnp.float32)
    zr_ref[...] = zr

    idx_cols = []
    fidx_rows = []
    for m in range(M):
        zs = zr[:, m * Ds:(m + 1) * Ds]
        zn = zs / jnp.maximum(
            jnp.sqrt(jnp.sum(zs * zs, axis=-1, keepdims=True)), _EPS)
        cn = cn_ref[m * K:(m + 1) * K, :]
        sim = jax.lax.dot_general(
            zn, cn, (((1,), (1,)), ((), ())),
            preferred_element_type=jnp.float32)
        idx = jnp.argmin(1.0 - sim, axis=-1).astype(jnp.int32)
        idx_cols.append(idx[:, None])
        fidx_rows.append(idx[None, :] + (m * K))

    idx_ref[...] = jnp.concatenate(idx_cols, axis=1)
    fidx_ref[...] = jnp.concatenate(fidx_rows, axis=0)


def _tc_finish_body(zr_ref, q0_ref, q1_ref, q2_ref, q3_ref, w_ref, zq_ref,
                    commit_ref):
    i = pl.program_id(0)
    Ds = 64
    zr = zr_ref[...]
    w = w_ref[...]
    q = jnp.concatenate(
        [q0_ref[:, :Ds], q1_ref[:, :Ds], q2_ref[:, :Ds], q3_ref[:, :Ds]],
        axis=1)
    st = zr + (q - zr)
    zq_ref[...] = jax.lax.dot_general(
        st, w, (((1,), (1,)), ((), ())),
        preferred_element_type=jnp.float32)

    diff = zr - q
    s = jnp.sum(diff * diff)

    @pl.when(i == 0)
    def _():
        commit_ref[0, 0] = s

    @pl.when(i > 0)
    def _():
        commit_ref[0, 0] += s


def _make_sc_gather(rows):
    lanes = 128
    info = plsc.get_sparse_core_info()
    nw = info.num_cores * info.num_subcores
    b_per_w = rows // nw
    chunk = 128
    n_chunks = b_per_w // chunk
    mesh = plsc.VectorSubcoreMesh(core_axis_name="c", subcore_axis_name="s")

    @functools.partial(
        pl.kernel, mesh=mesh,
        out_type=jax.ShapeDtypeStruct((rows, lanes), jnp.float32),
        scratch_types=[
            pltpu.VMEM((b_per_w,), jnp.int32),
            pltpu.VMEM((chunk, lanes), jnp.float32),
            pltpu.VMEM((chunk, lanes), jnp.float32),
            pltpu.SemaphoreType.DMA,
            pltpu.SemaphoreType.DMA,
        ],
    )
    def gather_kernel(cb_hbm, fidx_hbm, out_hbm, idx_v, rows_a, rows_b, sem_a,
                      sem_b):
        wid = lax.axis_index("s") * info.num_cores + lax.axis_index("c")
        base = wid * b_per_w
        pltpu.sync_copy(fidx_hbm.at[pl.ds(base, b_per_w)], idx_v)
        bufs = ((rows_a, sem_a), (rows_b, sem_b))
        copies = [None, None]
        for ci in range(n_chunks):
            buf, sem = bufs[ci % 2]
            copies[ci % 2] = pltpu.async_copy(
                cb_hbm.at[idx_v.at[pl.ds(ci * chunk, chunk)]], buf, sem)
            if ci >= 1:
                pbuf, _ = bufs[(ci - 1) % 2]
                copies[(ci - 1) % 2].wait()
                pltpu.sync_copy(
                    pbuf, out_hbm.at[pl.ds(base + (ci - 1) * chunk, chunk)])
        lbuf, _ = bufs[(n_chunks - 1) % 2]
        copies[(n_chunks - 1) % 2].wait()
        pltpu.sync_copy(
            lbuf, out_hbm.at[pl.ds(base + (n_chunks - 1) * chunk, chunk)])

    return gather_kernel


def kernel(z, W, codebooks):
    B, T, D = z.shape
    M, K, Ds = codebooks.shape
    N = B * T
    TN = 512
    grid = N // TN

    z_flat = z.reshape(N, D)
    cb_flat = codebooks.reshape(M * K, Ds)
    cb_pad = jnp.pad(cb_flat, ((0, 0), (0, 128 - Ds)))

    zr, idx, fidx = pl.pallas_call(
        _tc_assign_body,
        grid=(grid,),
        in_specs=[
            pl.BlockSpec((TN, D), lambda i: (i, 0)),
            pl.BlockSpec((D, D), lambda i: (0, 0)),
            pl.BlockSpec((M * K, Ds), lambda i: (0, 0)),
        ],
        out_specs=[
            pl.BlockSpec((TN, D), lambda i: (i, 0)),
            pl.BlockSpec((TN, M), lambda i: (i, 0)),
            pl.BlockSpec((M, TN), lambda i: (0, i)),
        ],
        out_shape=[
            jax.ShapeDtypeStruct((N, D), jnp.float32),
            jax.ShapeDtypeStruct((N, M), jnp.int32),
            jax.ShapeDtypeStruct((M, N), jnp.int32),
        ],
        scratch_shapes=[pltpu.VMEM((M * K, Ds), jnp.float32)],
        compiler_params=pltpu.CompilerParams(
            dimension_semantics=("arbitrary",)),
    )(z_flat, W, cb_flat)

    gather = _make_sc_gather(M * N)
    q = gather(cb_pad, fidx.reshape(M * N))

    nb = N // TN
    q_specs = [
        pl.BlockSpec((TN, 128), functools.partial(
            lambda m, i: (m * nb + i, 0), m)) for m in range(M)
    ]
    zq, commit = pl.pallas_call(
        _tc_finish_body,
        grid=(grid,),
        in_specs=[pl.BlockSpec((TN, D), lambda i: (i, 0))] + q_specs + [
            pl.BlockSpec((D, D), lambda i: (0, 0)),
        ],
        out_specs=[
            pl.BlockSpec((TN, D), lambda i: (i, 0)),
            pl.BlockSpec((1, 1), lambda i: (0, 0), memory_space=pltpu.SMEM),
        ],
        out_shape=[
            jax.ShapeDtypeStruct((N, D), jnp.float32),
            jax.ShapeDtypeStruct((1, 1), jnp.float32),
        ],
        compiler_params=pltpu.CompilerParams(
            dimension_semantics=("arbitrary",)),
    )(zr, q, q, q, q, W)

    return (zq.reshape(B, T, D), idx.reshape(B, T, M),
            commit[0, 0] / jnp.float32(N * D))


# speedup vs baseline: 1.4785x; 1.4785x over previous
import jax
import jax.numpy as jnp
from jax.experimental import pallas as pl
from jax.experimental.pallas import tpu as pltpu

_EPS = 1e-12


def _tc_body(z_ref, w_ref, cb_ref, zq_ref, idx_ref, commit_ref, cn_ref):
    i = pl.program_id(0)
    tn = z_ref.shape[0]
    M = 4
    K = 1024
    Ds = 64

    @pl.when(i == 0)
    def _():
        cb = cb_ref[...]
        cn_ref[...] = cb / jnp.maximum(
            jnp.sqrt(jnp.sum(cb * cb, axis=-1, keepdims=True)), _EPS)

    z = z_ref[...]
    w = w_ref[...]
    zr = jnp.dot(z, w, preferred_element_type=jnp.float32)

    idx_cols = []
    q_parts = []
    for m in range(M):
        zs = zr[:, m * Ds:(m + 1) * Ds]
        zn = zs / jnp.maximum(
            jnp.sqrt(jnp.sum(zs * zs, axis=-1, keepdims=True)), _EPS)
        cn = cn_ref[m * K:(m + 1) * K, :]
        sim = jax.lax.dot_general(
            zn, cn, (((1,), (1,)), ((), ())),
            preferred_element_type=jnp.float32)
        idx = jnp.argmin(1.0 - sim, axis=-1).astype(jnp.int32)
        oh = (jax.lax.broadcasted_iota(jnp.int32, (tn, K), 1)
              == idx[:, None]).astype(jnp.float32)
        cm = cb_ref[m * K:(m + 1) * K, :]
        qm = jnp.dot(oh, cm, preferred_element_type=jnp.float32)
        idx_cols.append(idx[:, None])
        q_parts.append(qm)

    zq_rot = jnp.concatenate(q_parts, axis=1)
    idx_ref[...] = jnp.concatenate(idx_cols, axis=1)

    st = zr + (zq_rot - zr)
    zq_ref[...] = jax.lax.dot_general(
        st, w, (((1,), (1,)), ((), ())),
        preferred_element_type=jnp.float32)

    diff = zr - zq_rot
    s = jnp.sum(diff * diff)

    @pl.when(i == 0)
    def _():
        commit_ref[0, 0] = s

    @pl.when(i > 0)
    def _():
        commit_ref[0, 0] += s


def kernel(z, W, codebooks):
    B, T, D = z.shape
    M, K, Ds = codebooks.shape
    N = B * T
    TN = 1024
    grid = N // TN

    z_flat = z.reshape(N, D)
    cb_flat = codebooks.reshape(M * K, Ds)

    zq, idx, commit = pl.pallas_call(
        _tc_body,
        grid=(grid,),
        in_specs=[
            pl.BlockSpec((TN, D), lambda i: (i, 0)),
            pl.BlockSpec((D, D), lambda i: (0, 0)),
            pl.BlockSpec((M * K, Ds), lambda i: (0, 0)),
        ],
        out_specs=[
            pl.BlockSpec((TN, D), lambda i: (i, 0)),
            pl.BlockSpec((TN, M), lambda i: (i, 0)),
            pl.BlockSpec((1, 1), lambda i: (0, 0), memory_space=pltpu.SMEM),
        ],
        out_shape=[
            jax.ShapeDtypeStruct((N, D), jnp.float32),
            jax.ShapeDtypeStruct((N, M), jnp.int32),
            jax.ShapeDtypeStruct((1, 1), jnp.float32),
        ],
        scratch_shapes=[pltpu.VMEM((M * K, Ds), jnp.float32)],
        compiler_params=pltpu.CompilerParams(
            dimension_semantics=("arbitrary",)),
    )(z_flat, W, cb_flat)

    return (zq.reshape(B, T, D), idx.reshape(B, T, M),
            commit[0, 0] / jnp.float32(N * D))


# speedup vs baseline: 1.8198x; 1.2308x over previous
import jax
import jax.numpy as jnp
from jax.experimental import pallas as pl
from jax.experimental.pallas import tpu as pltpu

_EPS = 1e-12


def _tc_body(z_ref, w_ref, cb_ref, zq_ref, idx_ref, commit_ref, cn_ref):
    i = pl.program_id(0)
    tn = z_ref.shape[0]
    M = 4
    K = 1024
    Ds = 64
    D = M * Ds

    @pl.when(i == 0)
    def _():
        cb = cb_ref[...]
        cn_ref[...] = cb / jnp.maximum(
            jnp.sqrt(jnp.sum(cb * cb, axis=-1, keepdims=True)), _EPS)

    z = z_ref[...]
    w = w_ref[...]
    zr = jnp.dot(z, w, preferred_element_type=jnp.float32)

    rows = jax.lax.broadcasted_iota(jnp.int32, (D, M), 0)
    cols = jax.lax.broadcasted_iota(jnp.int32, (D, M), 1)
    bd = jnp.where(rows // Ds == cols, 1.0, 0.0).astype(jnp.float32)
    zz = zr * zr
    norms = jnp.maximum(
        jnp.sqrt(jnp.dot(zz, bd, preferred_element_type=jnp.float32)),
        _EPS)

    idx_cols = []
    q_parts = []
    for m in range(M):
        zs = zr[:, m * Ds:(m + 1) * Ds]
        zn = zs / norms[:, m][:, None]
        cn = cn_ref[m * K:(m + 1) * K, :]
        sim = jax.lax.dot_general(
            zn, cn, (((1,), (1,)), ((), ())),
            preferred_element_type=jnp.float32)
        idx = jnp.argmax(sim, axis=-1).astype(jnp.int32)
        oh = (jax.lax.broadcasted_iota(jnp.int32, (tn, K), 1)
              == idx[:, None]).astype(jnp.float32)
        cm = cb_ref[m * K:(m + 1) * K, :]
        qm = jnp.dot(oh, cm, preferred_element_type=jnp.float32)
        idx_cols.append(idx[:, None])
        q_parts.append(qm)

    zq_rot = jnp.concatenate(q_parts, axis=1)
    idx_ref[...] = jnp.concatenate(idx_cols, axis=1)

    st = zr + (zq_rot - zr)
    zq_ref[...] = jax.lax.dot_general(
        st, w, (((1,), (1,)), ((), ())),
        preferred_element_type=jnp.float32)

    diff = zr - zq_rot
    s = jnp.sum(diff * diff)

    @pl.when(i == 0)
    def _():
        commit_ref[0, 0] = s

    @pl.when(i > 0)
    def _():
        commit_ref[0, 0] += s


def kernel(z, W, codebooks):
    B, T, D = z.shape
    M, K, Ds = codebooks.shape
    N = B * T
    TN = 1024
    grid = N // TN

    z_flat = z.reshape(N, D)
    cb_flat = codebooks.reshape(M * K, Ds)

    zq, idx, commit = pl.pallas_call(
        _tc_body,
        grid=(grid,),
        in_specs=[
            pl.BlockSpec((TN, D), lambda i: (i, 0)),
            pl.BlockSpec((D, D), lambda i: (0, 0)),
            pl.BlockSpec((M * K, Ds), lambda i: (0, 0)),
        ],
        out_specs=[
            pl.BlockSpec((TN, D), lambda i: (i, 0)),
            pl.BlockSpec((TN, M), lambda i: (i, 0)),
            pl.BlockSpec((1, 1), lambda i: (0, 0), memory_space=pltpu.SMEM),
        ],
        out_shape=[
            jax.ShapeDtypeStruct((N, D), jnp.float32),
            jax.ShapeDtypeStruct((N, M), jnp.int32),
            jax.ShapeDtypeStruct((1, 1), jnp.float32),
        ],
        scratch_shapes=[pltpu.VMEM((M * K, Ds), jnp.float32)],
        compiler_params=pltpu.CompilerParams(
            dimension_semantics=("arbitrary",)),
    )(z_flat, W, cb_flat)

    return (zq.reshape(B, T, D), idx.reshape(B, T, M),
            commit[0, 0] / jnp.float32(N * D))
